# baseline (speedup 1.0000x reference)
import functools

import jax
import jax.numpy as jnp
from jax import lax
from jax.experimental import pallas as pl
from jax.experimental.pallas import tpu as pltpu
from jax.experimental.pallas import tpu_sc as plsc

VOCAB = 1000000
DIM = 64
BATCH = 4096
HIST = 50
N = BATCH * HIST
NSPLIT = 2
BSPLIT = BATCH // NSPLIT

_info = plsc.get_sparse_core_info()
_NC = _info.num_cores
_NS = _info.num_subcores
NW = _NC * _NS
B_PER_W = BSPLIT // NW
ROWS_PER_W = B_PER_W * HIST
CB = 8
CR = CB * HIST
NCHUNK = B_PER_W // CB

_mesh = plsc.VectorSubcoreMesh(core_axis_name="c", subcore_axis_name="s")


@functools.partial(
    pl.kernel,
    mesh=_mesh,
    out_type=jax.ShapeDtypeStruct((BSPLIT, HIST, DIM), jnp.float32),
    scratch_types=[
        pltpu.VMEM((ROWS_PER_W,), jnp.int32),
        pltpu.VMEM((2, CR, DIM), jnp.float32),
        pltpu.SemaphoreType.DMA,
        pltpu.SemaphoreType.DMA,
        pltpu.SemaphoreType.DMA,
        pltpu.SemaphoreType.DMA,
    ],
)
def _emb_lookup(ids_hbm, table_hbm, out_hbm, idx_v, buf, g0, g1, o0, o1):
    wid = lax.axis_index("s") * _NC + lax.axis_index("c")
    base_row = wid * ROWS_PER_W
    base_batch = wid * B_PER_W
    pltpu.sync_copy(ids_hbm.at[pl.ds(base_row, ROWS_PER_W)], idx_v)
    gsems = (g0, g1)
    osems = (o0, o1)

    def issue_chunk(g, slot):
        def body(t, carry):
            v = idx_v[pl.ds(g * CR + t * 16, 16)]
            for u in range(16):
                j = t * 16 + u
                pltpu.async_copy(
                    table_hbm.at[
                        lax.shift_right_logical(v[u], 3), v[u] & 7
                    ],
                    buf.at[slot, j],
                    gsems[slot],
                )
            return carry
        lax.fori_loop(0, CR // 16, body, 0)

    def drain_chunk(slot):
        pltpu.make_async_copy(
            out_hbm.at[0], buf.at[slot, pl.ds(0, HIST)], gsems[slot]
        ).wait()

        def extra(b, carry):
            pltpu.make_async_copy(
                out_hbm.at[0], buf.at[slot, pl.ds(0, HIST)], gsems[slot]
            ).wait()
            return carry

        lax.fori_loop(0, CB - 1, extra, 0)

    def write_chunk(g, slot):
        for b in range(CB):
            pltpu.async_copy(
                buf.at[slot, pl.ds(b * HIST, HIST)],
                out_hbm.at[base_batch + g * CB + b],
                osems[slot],
            )

    def wait_write(slot):
        def step(b, carry):
            pltpu.make_async_copy(
                buf.at[slot, pl.ds(0, HIST)], out_hbm.at[0], osems[slot]
            ).wait()
            return carry

        lax.fori_loop(0, CB, step, 0)

    issue_chunk(0, 0)
    drain_chunk(0)
    write_chunk(0, 0)
    for g in range(1, NCHUNK):
        slot = g % 2
        if g >= 2:
            wait_write(slot)
        issue_chunk(g, slot)
        drain_chunk(slot)
        write_chunk(g, slot)
    wait_write(0)
    wait_write(1)


def kernel(input_ids, table):
    ids = input_ids.reshape(-1).astype(jnp.int32)
    t3 = table.reshape(VOCAB // 8, 8, DIM)
    halves = [
        _emb_lookup(ids[i * BSPLIT * HIST:(i + 1) * BSPLIT * HIST], t3)
        for i in range(NSPLIT)
    ]
    return jnp.concatenate(halves, axis=0)


# speedup vs baseline: 1.0593x; 1.0593x over previous
import functools

import jax
import jax.numpy as jnp
from jax import lax
from jax.experimental import pallas as pl
from jax.experimental.pallas import tpu as pltpu
from jax.experimental.pallas import tpu_sc as plsc

VOCAB = 1000000
DIM = 64
BATCH = 4096
HIST = 50
N = BATCH * HIST

_info = plsc.get_sparse_core_info()
_NC = _info.num_cores
_NS = _info.num_subcores
NW = _NC * _NS
B_PER_W = BATCH // NW
ROWS_PER_W = B_PER_W * HIST
CB = 8
CR = CB * HIST
NCHUNK = B_PER_W // CB

_mesh = plsc.VectorSubcoreMesh(core_axis_name="c", subcore_axis_name="s")


@functools.partial(
    pl.kernel,
    mesh=_mesh,
    out_type=jax.ShapeDtypeStruct((BATCH, HIST, DIM), jnp.float32),
    scratch_types=[
        pltpu.VMEM((ROWS_PER_W,), jnp.int32),
        pltpu.VMEM((2, CR, DIM), jnp.float32),
        pltpu.SemaphoreType.DMA,
        pltpu.SemaphoreType.DMA,
        pltpu.SemaphoreType.DMA,
        pltpu.SemaphoreType.DMA,
    ],
)
def _emb_lookup(ids_hbm, table_hbm, out_hbm, idx_v, buf, g0, g1, o0, o1):
    wid = lax.axis_index("s") * _NC + lax.axis_index("c")
    base_row = wid * ROWS_PER_W
    base_batch = wid * B_PER_W
    pltpu.sync_copy(ids_hbm.at[pl.ds(base_row, ROWS_PER_W)], idx_v)
    gsems = (g0, g1)
    osems = (o0, o1)

    def issue_chunk(g, slot):
        def body(t, carry):
            v = idx_v[pl.ds(g * CR + t * 16, 16)]
            for u in range(16):
                j = t * 16 + u
                pltpu.async_copy(
                    table_hbm.at[
                        lax.shift_right_logical(v[u], 3), v[u] & 7
                    ],
                    buf.at[slot, j],
                    gsems[slot],
                )
            return carry
        lax.fori_loop(0, CR // 16, body, 0)

    def drain_chunk(slot):
        pltpu.make_async_copy(
            out_hbm.at[0], buf.at[slot, pl.ds(0, HIST)], gsems[slot]
        ).wait()

        def extra(b, carry):
            pltpu.make_async_copy(
                out_hbm.at[0], buf.at[slot, pl.ds(0, HIST)], gsems[slot]
            ).wait()
            return carry

        lax.fori_loop(0, CB - 1, extra, 0)

    def write_chunk(g, slot):
        for b in range(CB):
            pltpu.async_copy(
                buf.at[slot, pl.ds(b * HIST, HIST)],
                out_hbm.at[base_batch + g * CB + b],
                osems[slot],
            )

    def wait_write(slot):
        def step(b, carry):
            pltpu.make_async_copy(
                buf.at[slot, pl.ds(0, HIST)], out_hbm.at[0], osems[slot]
            ).wait()
            return carry

        lax.fori_loop(0, CB, step, 0)

    issue_chunk(0, 0)
    drain_chunk(0)
    write_chunk(0, 0)
    for g in range(1, NCHUNK):
        slot = g % 2
        if g >= 2:
            wait_write(slot)
        issue_chunk(g, slot)
        drain_chunk(slot)
        write_chunk(g, slot)
    wait_write(0)
    wait_write(1)


def kernel(input_ids, table):
    ids = input_ids.reshape(-1).astype(jnp.int32)
    t3 = table.reshape(VOCAB // 8, 8, DIM)
    return _emb_lookup(ids, t3)
